# baseline (speedup 1.0000x reference)
import functools

import jax
import jax.numpy as jnp
import numpy as np
from jax import lax
from jax.experimental import pallas as pl
from jax.experimental.pallas import tpu as pltpu
from jax.experimental.pallas import tpu_sc as plsc

FTINY = float(np.finfo(np.float32).tiny) * 1e3
INF_VALUE = float(np.finfo(np.float32).max) * 1e-3
LOWER_INF = float(np.finfo(np.float32).max) * 1e-4

HEIGHT = 224
WIDTH = 224
N_FACES = 4000
N_VERTS = 2100

NW = 32
L = 16
FPAD = 4096
VPAD = 2112
FPW = FPAD // NW
ROWS = HEIGHT // NW
NCHUNK = WIDTH // L
BAND = ROWS * WIDTH


def _wid():
    return lax.axis_index("s") * 2 + lax.axis_index("c")


def _face_setup_body(vert_hbm, face_hbm, nrm_hbm, rt_hbm,
                     recf_hbm, colc_hbm,
                     vert_v, face_v, nrm_v, rt_v, stf_v, stc_v):
    w = _wid()
    base = w * FPW
    pltpu.sync_copy(vert_hbm, vert_v)
    pltpu.sync_copy(rt_hbm, rt_v)
    for j in range(3):
        pltpu.sync_copy(face_hbm.at[pl.ds(j * FPAD + base, FPW)],
                        face_v.at[pl.ds(j * FPW, FPW)])
        pltpu.sync_copy(nrm_hbm.at[pl.ds(j * FPAD + base, FPW)],
                        nrm_v.at[pl.ds(j * FPW, FPW)])
    rtv = rt_v[pl.ds(0, L)]
    rtx = rtv[0]
    rty = rtv[1]
    rtz = rtv[2]
    iota = lax.iota(jnp.int32, L)

    for k in range(FPW // L):
        o = k * L
        i0 = face_v[pl.ds(o, L)]
        i1 = face_v[pl.ds(FPW + o, L)]
        i2 = face_v[pl.ds(2 * FPW + o, L)]

        def gath(row, idx):
            return plsc.load_gather(vert_v, [jnp.int32(row * VPAD) + idx])

        x0 = gath(0, i0); y0 = gath(1, i0); z0 = gath(2, i0)
        x1 = gath(0, i1); y1 = gath(1, i1); z1 = gath(2, i1)
        x2 = gath(0, i2); y2 = gath(1, i2); z2 = gath(2, i2)

        p3x = gath(6, i0); p3y = gath(7, i0); p3z = gath(8, i0)
        nx = nrm_v[pl.ds(o, L)]
        ny = nrm_v[pl.ds(FPW + o, L)]
        nz = nrm_v[pl.ds(2 * FPW + o, L)]
        ndot = (p3x + rtx) * nx + (p3y + rty) * ny + (p3z + rtz) * nz
        zmin = jnp.minimum(jnp.minimum(z0, z1), z2)
        fid = base + o + iota
        valid = (ndot < 0.0) & (zmin > 0.0) & (fid < N_FACES)

        det = (y1 - y2) * (x0 - x2) + (x2 - x1) * (y0 - y2)
        det = jnp.sign(det) * jnp.maximum(jnp.abs(det), FTINY)
        inv = 1.0 / det
        l0x = (y1 - y2) * inv
        l0y = (x2 - x1) * inv
        l0c = -l0x * x2 - l0y * y2
        l1x = (y2 - y0) * inv
        l1y = (x0 - x2) * inv
        l1c = -l1x * x2 - l1y * y2
        l2x = -l0x - l1x
        l2y = -l0y - l1y
        l2c = 1.0 - l0c - l1c
        dx = z0 * l0x + z1 * l1x + z2 * l2x
        dy = z0 * l0y + z1 * l1y + z2 * l2y
        dc = z0 * l0c + z1 * l1c + z2 * l2c

        rec_off = (o + iota) * 16

        for ch in range(3):
            c0 = gath(3 + ch, i0)
            c1 = gath(3 + ch, i1)
            c2 = gath(3 + ch, i2)
            stc_v[pl.ds((3 * ch + 0) * FPW + o, L)] = c0 * l0x + c1 * l1x + c2 * l2x
            stc_v[pl.ds((3 * ch + 1) * FPW + o, L)] = c0 * l0y + c1 * l1y + c2 * l2y
            stc_v[pl.ds((3 * ch + 2) * FPW + o, L)] = c0 * l0c + c1 * l1c + c2 * l2c

        BIG = 1e9
        span_a, span_b, span_lo, span_hi = [], [], [], []
        for (ex, ey, ec) in ((l0x, l0y, l0c), (l1x, l1y, l1c), (l2x, l2y, l2c)):
            ai = -ec / ex
            bi = -ey / ex
            mag = jnp.abs(ai) + 224.0 * jnp.abs(bi)
            mi = mag * 4e-3 + 1.0
            sane = mag < BIG
            is_lo = (ex > 0.0) & sane
            is_hi = (ex < 0.0) & sane
            span_a.append(jnp.where(is_lo, ai - mi, jnp.where(is_hi, ai + mi, 0.0)))
            span_b.append(jnp.where(is_lo | is_hi, bi, 0.0))
            span_lo.append(is_lo)
            span_hi.append(is_hi)

        ymn = jnp.minimum(jnp.minimum(y0, y1), y2)
        ymx = jnp.maximum(jnp.maximum(y0, y1), y2)
        ymin_i = jnp.clip((ymn - 1.0).astype(jnp.int32), 0, HEIGHT - 1)
        ymax_i = jnp.clip(ymx.astype(jnp.int32) + 1, 0, HEIGHT - 1)
        ymin_i = jnp.where(valid, ymin_i, 1)
        ymax_i = jnp.where(valid, ymax_i, 0)
        packed = ymin_i | (ymax_i << 9)
        for e in range(3):
            packed = (packed
                      | (span_lo[e].astype(jnp.int32) << (18 + 2 * e))
                      | (span_hi[e].astype(jnp.int32) << (19 + 2 * e)))

        vals = (l0x, l0y, l0c, l1x, l1y, l1c, dx, dy, dc,
                span_a[0], span_b[0], span_a[1], span_b[1],
                span_a[2], span_b[2],
                plsc.bitcast(packed, jnp.float32))
        for j, v in enumerate(vals):
            plsc.store_scatter(stf_v, [rec_off + j], v)

    pltpu.sync_copy(stf_v, recf_hbm.at[pl.ds(base * 16, FPW * 16)])
    for j in range(9):
        pltpu.sync_copy(stc_v.at[pl.ds(j * FPW, FPW)],
                        colc_hbm.at[pl.ds(j * FPAD + base, FPW)])


def _raster_body(recf_hbm, colc_hbm,
                 img_hbm, msk_hbm,
                 recf_v, colc_v, zbuf_v, ibuf_v, img_v, msk_v):
    w = _wid()
    band_y0 = w * ROWS
    pltpu.sync_copy(recf_hbm, recf_v)
    pltpu.sync_copy(colc_hbm, colc_v)

    inf_v = jnp.full((L,), INF_VALUE, jnp.float32)
    zero_i = jnp.zeros((L,), jnp.int32)
    for k in range(BAND // L):
        zbuf_v[pl.ds(k * L, L)] = inf_v
        ibuf_v[pl.ds(k * L, L)] = zero_i

    iota_f = lax.iota(jnp.int32, L).astype(jnp.float32)

    def face_body(f, carry):
        rec = recf_v[pl.ds(f * 16, L)]
        reci = plsc.bitcast(rec, jnp.int32)
        pk = reci[15]
        ymin = pk & 511
        ymax = (pk >> 9) & 511
        ylo = jnp.maximum(ymin, band_y0)
        yhi = jnp.minimum(ymax, band_y0 + (ROWS - 1))

        @pl.when(ylo <= yhi)
        def _process():
            def bcast(j):
                return jnp.broadcast_to(rec[j], (L,))

            l0xv = bcast(0)
            l0yv = bcast(1)
            l0cv = bcast(2)
            l1xv = bcast(3)
            l1yv = bcast(4)
            l1cv = bcast(5)
            l2xv = -l0xv - l1xv
            l2yv = -l0yv - l1yv
            l2cv = 1.0 - l0cv - l1cv
            dxv = bcast(6)
            dyv = bcast(7)
            dcv = bcast(8)
            sa0 = rec[9]
            sb0 = rec[10]
            sa1 = rec[11]
            sb1 = rec[12]
            sa2 = rec[13]
            sb2 = rec[14]
            lo0 = (pk & (1 << 18)) != 0
            hi0 = (pk & (1 << 19)) != 0
            lo1 = (pk & (1 << 20)) != 0
            hi1 = (pk & (1 << 21)) != 0
            lo2 = (pk & (1 << 22)) != 0
            hi2 = (pk & (1 << 23)) != 0

            def row_body(y, _):
                yf = y.astype(jnp.float32)
                xe0 = sa0 + sb0 * yf
                xe1 = sa1 + sb1 * yf
                xe2 = sa2 + sb2 * yf
                xlo = jnp.maximum(
                    jnp.maximum(jnp.where(lo0, xe0, 0.0),
                                jnp.where(lo1, xe1, 0.0)),
                    jnp.where(lo2, xe2, 0.0))
                xhi = jnp.minimum(
                    jnp.minimum(jnp.where(hi0, xe0, float(WIDTH - 1)),
                                jnp.where(hi1, xe1, float(WIDTH - 1))),
                    jnp.where(hi2, xe2, float(WIDTH - 1)))

                v0 = xlo * (1.0 / (2 * L))
                v1 = xhi * (1.0 / (2 * L))
                t0 = v0.astype(jnp.int32)
                t1 = v1.astype(jnp.int32)
                t0 = t0 - (t0.astype(jnp.float32) > v0).astype(jnp.int32)
                t1 = t1 - (t1.astype(jnp.float32) > v1).astype(jnp.int32)
                p0r = jnp.clip(t0, 0, NCHUNK // 2 - 1)
                p1r = jnp.clip(t1, 0, NCHUNK // 2 - 1)
                a0v = l0yv * yf + l0cv
                a1v = l1yv * yf + l1cv
                a2v = l2yv * yf + l2cv
                adv = dyv * yf + dcv
                row_off = (y - band_y0) * WIDTH

                def pair_body(p, _c):
                    xb = (p * (2 * L)).astype(jnp.float32)
                    for half in range(2):
                        xv = xb + (iota_f + float(half * L))
                        lv0 = l0xv * xv + a0v
                        lv1 = l1xv * xv + a1v
                        lv2 = l2xv * xv + a2v
                        m = jnp.minimum(jnp.minimum(lv0, lv1), lv2) >= 0.0
                        d = dxv * xv + adv
                        off = row_off + p * (2 * L) + half * L
                        z = zbuf_v[pl.ds(off, L)]
                        upd = m & (d < z)
                        zbuf_v[pl.ds(off, L)] = jnp.where(upd, d, z)
                        ib = ibuf_v[pl.ds(off, L)]
                        ibuf_v[pl.ds(off, L)] = jnp.where(upd, f, ib)
                    return 0

                lax.fori_loop(p0r, p1r + 1, pair_body, 0)

                return 0

            lax.fori_loop(ylo, yhi + 1, row_body, 0)

        return carry

    lax.fori_loop(0, FPAD, face_body, 0)

    def res_body(k, _):
        off = k * L
        z = zbuf_v[pl.ds(off, L)]
        idx = ibuf_v[pl.ds(off, L)]
        mskf = (z < LOWER_INF).astype(jnp.float32)
        y = band_y0 + k // NCHUNK
        yf = y.astype(jnp.float32)
        xv = ((k % NCHUNK) * L).astype(jnp.float32) + iota_f
        for ch in range(3):
            cx = plsc.load_gather(colc_v, [jnp.int32((3 * ch + 0) * FPAD) + idx])
            cy = plsc.load_gather(colc_v, [jnp.int32((3 * ch + 1) * FPAD) + idx])
            cc = plsc.load_gather(colc_v, [jnp.int32((3 * ch + 2) * FPAD) + idx])
            img_v[pl.ds(ch * BAND + off, L)] = mskf * (cx * xv + cy * yf + cc)
        msk_v[pl.ds(off, L)] = mskf
        return 0

    lax.fori_loop(0, BAND // L, res_body, 0)

    for ch in range(3):
        pltpu.sync_copy(img_v.at[pl.ds(ch * BAND, BAND)],
                        img_hbm.at[pl.ds(ch * (HEIGHT * WIDTH) + w * BAND, BAND)])
    pltpu.sync_copy(msk_v, msk_hbm.at[pl.ds(w * BAND, BAND)])


def _build_mesh():
    return plsc.VectorSubcoreMesh(core_axis_name="c", subcore_axis_name="s")


@jax.jit
def _rasterize(vert, facep, nrm, rt):
    mesh = _build_mesh()
    f32 = jnp.float32
    i32 = jnp.int32
    cparams = pltpu.CompilerParams(needs_layout_passes=False)
    setup = pl.kernel(
        _face_setup_body,
        compiler_params=cparams,
        out_type=(
            jax.ShapeDtypeStruct((16 * FPAD,), f32),
            jax.ShapeDtypeStruct((9 * FPAD,), f32),
        ),
        mesh=mesh,
        scratch_types=[
            pltpu.VMEM((9 * VPAD,), f32),
            pltpu.VMEM((3 * FPW,), i32),
            pltpu.VMEM((3 * FPW,), f32),
            pltpu.VMEM((16,), f32),
            pltpu.VMEM((16 * FPW,), f32),
            pltpu.VMEM((9 * FPW,), f32),
        ],
    )
    recf, colc = setup(vert, facep, nrm, rt)
    raster = pl.kernel(
        _raster_body,
        compiler_params=cparams,
        out_type=(
            jax.ShapeDtypeStruct((3 * HEIGHT * WIDTH,), f32),
            jax.ShapeDtypeStruct((HEIGHT * WIDTH,), f32),
        ),
        mesh=mesh,
        scratch_types=[
            pltpu.VMEM((16 * FPAD,), f32),
            pltpu.VMEM((9 * FPAD,), f32),
            pltpu.VMEM((BAND,), f32),
            pltpu.VMEM((BAND,), i32),
            pltpu.VMEM((3 * BAND,), f32),
            pltpu.VMEM((BAND,), f32),
        ],
    )
    img, msk = raster(recf, colc)
    return img, msk


def kernel(pt_2d, color, pt_3d, normal, R, T, face):
    batch = pt_2d.shape[0]
    images = []
    masks = []
    facep = jnp.zeros((3, FPAD), jnp.int32).at[:, :N_FACES].set(face).reshape(-1)
    for b in range(batch):
        vert = jnp.zeros((9, VPAD), jnp.float32)
        vert = vert.at[0:3, :N_VERTS].set(pt_2d[b])
        vert = vert.at[3:6, :N_VERTS].set(color[b])
        vert = vert.at[6:9, :N_VERTS].set(pt_3d[b])
        vert = vert.reshape(-1)
        nrm = (jnp.zeros((3, FPAD), jnp.float32)
               .at[:, :N_FACES].set(normal[b]).reshape(-1))
        rt = jnp.zeros((16,), jnp.float32).at[0:3].set((R[b].T @ T[b])[:, 0])
        img, msk = _rasterize(vert, facep, nrm, rt)
        images.append(img.reshape(3, HEIGHT, WIDTH))
        masks.append(msk.reshape(HEIGHT, WIDTH))
    image = jnp.stack(images, axis=0)
    mask = jnp.stack(masks, axis=0)
    return image, lax.stop_gradient(mask)


# speedup vs baseline: 1.0951x; 1.0951x over previous
import functools

import jax
import jax.numpy as jnp
import numpy as np
from jax import lax
from jax.experimental import pallas as pl
from jax.experimental.pallas import tpu as pltpu
from jax.experimental.pallas import tpu_sc as plsc

FTINY = float(np.finfo(np.float32).tiny) * 1e3
INF_VALUE = float(np.finfo(np.float32).max) * 1e-3
LOWER_INF = float(np.finfo(np.float32).max) * 1e-4

HEIGHT = 224
WIDTH = 224
N_FACES = 4000
N_VERTS = 2100

NW = 32
L = 16
FPAD = 4096
VPAD = 2112
FPW = FPAD // NW
ROWS = HEIGHT // NW
NCHUNK = WIDTH // L
BAND = ROWS * WIDTH


def _wid():
    return lax.axis_index("s") * 2 + lax.axis_index("c")


def _face_setup_body(vert_hbm, face_hbm, nrm_hbm, rt_hbm,
                     recf_hbm, colc_hbm,
                     vert_v, face_v, nrm_v, rt_v, stf_v, stc_v):
    w = _wid()
    base = w * FPW
    pltpu.sync_copy(vert_hbm, vert_v)
    pltpu.sync_copy(rt_hbm, rt_v)
    for j in range(3):
        pltpu.sync_copy(face_hbm.at[pl.ds(j * FPAD + base, FPW)],
                        face_v.at[pl.ds(j * FPW, FPW)])
        pltpu.sync_copy(nrm_hbm.at[pl.ds(j * FPAD + base, FPW)],
                        nrm_v.at[pl.ds(j * FPW, FPW)])
    rtv = rt_v[pl.ds(0, L)]
    rtx = rtv[0]
    rty = rtv[1]
    rtz = rtv[2]
    iota = lax.iota(jnp.int32, L)

    for k in range(FPW // L):
        o = k * L
        i0 = face_v[pl.ds(o, L)]
        i1 = face_v[pl.ds(FPW + o, L)]
        i2 = face_v[pl.ds(2 * FPW + o, L)]

        def gath(row, idx):
            return plsc.load_gather(vert_v, [jnp.int32(row * VPAD) + idx])

        x0 = gath(0, i0); y0 = gath(1, i0); z0 = gath(2, i0)
        x1 = gath(0, i1); y1 = gath(1, i1); z1 = gath(2, i1)
        x2 = gath(0, i2); y2 = gath(1, i2); z2 = gath(2, i2)

        p3x = gath(6, i0); p3y = gath(7, i0); p3z = gath(8, i0)
        nx = nrm_v[pl.ds(o, L)]
        ny = nrm_v[pl.ds(FPW + o, L)]
        nz = nrm_v[pl.ds(2 * FPW + o, L)]
        ndot = (p3x + rtx) * nx + (p3y + rty) * ny + (p3z + rtz) * nz
        zmin = jnp.minimum(jnp.minimum(z0, z1), z2)
        fid = base + o + iota
        valid = (ndot < 0.0) & (zmin > 0.0) & (fid < N_FACES)

        det = (y1 - y2) * (x0 - x2) + (x2 - x1) * (y0 - y2)
        det = jnp.sign(det) * jnp.maximum(jnp.abs(det), FTINY)
        inv = 1.0 / det
        l0x = (y1 - y2) * inv
        l0y = (x2 - x1) * inv
        l0c = -l0x * x2 - l0y * y2
        l1x = (y2 - y0) * inv
        l1y = (x0 - x2) * inv
        l1c = -l1x * x2 - l1y * y2
        l2x = -l0x - l1x
        l2y = -l0y - l1y
        l2c = 1.0 - l0c - l1c
        dx = z0 * l0x + z1 * l1x + z2 * l2x
        dy = z0 * l0y + z1 * l1y + z2 * l2y
        dc = z0 * l0c + z1 * l1c + z2 * l2c

        rec_off = (o + iota) * 16

        for ch in range(3):
            c0 = gath(3 + ch, i0)
            c1 = gath(3 + ch, i1)
            c2 = gath(3 + ch, i2)
            stc_v[pl.ds((3 * ch + 0) * FPW + o, L)] = c0 * l0x + c1 * l1x + c2 * l2x
            stc_v[pl.ds((3 * ch + 1) * FPW + o, L)] = c0 * l0y + c1 * l1y + c2 * l2y
            stc_v[pl.ds((3 * ch + 2) * FPW + o, L)] = c0 * l0c + c1 * l1c + c2 * l2c

        BIG = 1e9
        span_a, span_b, span_lo, span_hi = [], [], [], []
        for (ex, ey, ec) in ((l0x, l0y, l0c), (l1x, l1y, l1c), (l2x, l2y, l2c)):
            ai = -ec / ex
            bi = -ey / ex
            mag = jnp.abs(ai) + 224.0 * jnp.abs(bi)
            mi = mag * 4e-3 + 1.0
            sane = mag < BIG
            is_lo = (ex > 0.0) & sane
            is_hi = (ex < 0.0) & sane
            span_a.append(jnp.where(is_lo, ai - mi, jnp.where(is_hi, ai + mi, 0.0)))
            span_b.append(jnp.where(is_lo | is_hi, bi, 0.0))
            span_lo.append(is_lo)
            span_hi.append(is_hi)

        ymn = jnp.minimum(jnp.minimum(y0, y1), y2)
        ymx = jnp.maximum(jnp.maximum(y0, y1), y2)
        ymin_i = jnp.clip((ymn - 1.0).astype(jnp.int32), 0, HEIGHT - 1)
        ymax_i = jnp.clip(ymx.astype(jnp.int32) + 1, 0, HEIGHT - 1)
        ymin_i = jnp.where(valid, ymin_i, 1)
        ymax_i = jnp.where(valid, ymax_i, 0)
        packed = ymin_i | (ymax_i << 9)
        for e in range(3):
            packed = (packed
                      | (span_lo[e].astype(jnp.int32) << (18 + 2 * e))
                      | (span_hi[e].astype(jnp.int32) << (19 + 2 * e)))

        vals = (l0x, l0y, l0c, l1x, l1y, l1c, dx, dy, dc,
                span_a[0], span_b[0], span_a[1], span_b[1],
                span_a[2], span_b[2],
                plsc.bitcast(packed, jnp.float32))
        for j, v in enumerate(vals):
            plsc.store_scatter(stf_v, [rec_off + j], v)

    pltpu.sync_copy(stf_v, recf_hbm.at[pl.ds(base * 16, FPW * 16)])
    for j in range(9):
        pltpu.sync_copy(stc_v.at[pl.ds(j * FPW, FPW)],
                        colc_hbm.at[pl.ds(j * FPAD + base, FPW)])


def _raster_body(recf_hbm, colc_hbm,
                 img_hbm, msk_hbm,
                 recf_v, colc_v, zbuf_v, ibuf_v, img_v, msk_v):
    w = _wid()
    band_y0 = w * ROWS
    pltpu.sync_copy(recf_hbm, recf_v)
    pltpu.sync_copy(colc_hbm, colc_v)

    inf_v = jnp.full((L,), INF_VALUE, jnp.float32)
    zero_i = jnp.zeros((L,), jnp.int32)
    for k in range(BAND // L):
        zbuf_v[pl.ds(k * L, L)] = inf_v
        ibuf_v[pl.ds(k * L, L)] = zero_i

    iota_f = lax.iota(jnp.int32, L).astype(jnp.float32)

    def face_body(f, carry):
        rec = recf_v[pl.ds(f * 16, L)]
        reci = plsc.bitcast(rec, jnp.int32)
        pk = reci[15]
        ymin = pk & 511
        ymax = (pk >> 9) & 511
        ylo = jnp.maximum(ymin, band_y0)
        yhi = jnp.minimum(ymax, band_y0 + (ROWS - 1))

        @pl.when(ylo <= yhi)
        def _process():
            def bcast(j):
                return jnp.broadcast_to(rec[j], (L,))

            l0xv = bcast(0)
            l0yv = bcast(1)
            l0cv = bcast(2)
            l1xv = bcast(3)
            l1yv = bcast(4)
            l1cv = bcast(5)
            l2xv = -l0xv - l1xv
            l2yv = -l0yv - l1yv
            l2cv = 1.0 - l0cv - l1cv
            dxv = bcast(6)
            dyv = bcast(7)
            dcv = bcast(8)
            sa0 = rec[9]
            sb0 = rec[10]
            sa1 = rec[11]
            sb1 = rec[12]
            sa2 = rec[13]
            sb2 = rec[14]
            lo0 = (pk & (1 << 18)) != 0
            hi0 = (pk & (1 << 19)) != 0
            lo1 = (pk & (1 << 20)) != 0
            hi1 = (pk & (1 << 21)) != 0
            lo2 = (pk & (1 << 22)) != 0
            hi2 = (pk & (1 << 23)) != 0

            def row_body(y, _):
                yf = y.astype(jnp.float32)
                xe0 = sa0 + sb0 * yf
                xe1 = sa1 + sb1 * yf
                xe2 = sa2 + sb2 * yf
                xlo = jnp.maximum(
                    jnp.maximum(jnp.where(lo0, xe0, 0.0),
                                jnp.where(lo1, xe1, 0.0)),
                    jnp.where(lo2, xe2, 0.0))
                xhi = jnp.minimum(
                    jnp.minimum(jnp.where(hi0, xe0, float(WIDTH - 1)),
                                jnp.where(hi1, xe1, float(WIDTH - 1))),
                    jnp.where(hi2, xe2, float(WIDTH - 1)))

                v0 = xlo * (1.0 / L)
                v1 = xhi * (1.0 / L)
                t0 = v0.astype(jnp.int32)
                t1 = v1.astype(jnp.int32)
                t0 = t0 - (t0.astype(jnp.float32) > v0).astype(jnp.int32)
                t1 = t1 - (t1.astype(jnp.float32) > v1).astype(jnp.int32)
                c0r = jnp.clip(t0, 0, NCHUNK - 1)
                c1r = jnp.clip(t1, 0, NCHUNK - 1)
                a0v = l0yv * yf + l0cv
                a1v = l1yv * yf + l1cv
                a2v = l2yv * yf + l2cv
                adv = dyv * yf + dcv
                row_off = (y - band_y0) * WIDTH

                def chunk_body(cix, _c):
                    xv = (cix * L).astype(jnp.float32) + iota_f
                    lv0 = l0xv * xv + a0v
                    lv1 = l1xv * xv + a1v
                    lv2 = l2xv * xv + a2v
                    m = jnp.minimum(jnp.minimum(lv0, lv1), lv2) >= 0.0
                    d = dxv * xv + adv
                    off = row_off + cix * L
                    z = zbuf_v[pl.ds(off, L)]
                    upd = m & (d < z)
                    zbuf_v[pl.ds(off, L)] = jnp.where(upd, d, z)
                    ib = ibuf_v[pl.ds(off, L)]
                    ibuf_v[pl.ds(off, L)] = jnp.where(upd, f, ib)
                    return 0

                lax.fori_loop(c0r, c1r + 1, chunk_body, 0)

                return 0

            lax.fori_loop(ylo, yhi + 1, row_body, 0)

        return carry

    lax.fori_loop(0, FPAD, face_body, 0)

    def res_body(k, _):
        off = k * L
        z = zbuf_v[pl.ds(off, L)]
        idx = ibuf_v[pl.ds(off, L)]
        mskf = (z < LOWER_INF).astype(jnp.float32)
        y = band_y0 + k // NCHUNK
        yf = y.astype(jnp.float32)
        xv = ((k % NCHUNK) * L).astype(jnp.float32) + iota_f
        for ch in range(3):
            cx = plsc.load_gather(colc_v, [jnp.int32((3 * ch + 0) * FPAD) + idx])
            cy = plsc.load_gather(colc_v, [jnp.int32((3 * ch + 1) * FPAD) + idx])
            cc = plsc.load_gather(colc_v, [jnp.int32((3 * ch + 2) * FPAD) + idx])
            img_v[pl.ds(ch * BAND + off, L)] = mskf * (cx * xv + cy * yf + cc)
        msk_v[pl.ds(off, L)] = mskf
        return 0

    lax.fori_loop(0, BAND // L, res_body, 0)

    for ch in range(3):
        pltpu.sync_copy(img_v.at[pl.ds(ch * BAND, BAND)],
                        img_hbm.at[pl.ds(ch * (HEIGHT * WIDTH) + w * BAND, BAND)])
    pltpu.sync_copy(msk_v, msk_hbm.at[pl.ds(w * BAND, BAND)])


def _build_mesh():
    return plsc.VectorSubcoreMesh(core_axis_name="c", subcore_axis_name="s")


@jax.jit
def _rasterize(vert, facep, nrm, rt):
    mesh = _build_mesh()
    f32 = jnp.float32
    i32 = jnp.int32
    cparams = pltpu.CompilerParams(needs_layout_passes=False)
    setup = pl.kernel(
        _face_setup_body,
        compiler_params=cparams,
        out_type=(
            jax.ShapeDtypeStruct((16 * FPAD,), f32),
            jax.ShapeDtypeStruct((9 * FPAD,), f32),
        ),
        mesh=mesh,
        scratch_types=[
            pltpu.VMEM((9 * VPAD,), f32),
            pltpu.VMEM((3 * FPW,), i32),
            pltpu.VMEM((3 * FPW,), f32),
            pltpu.VMEM((16,), f32),
            pltpu.VMEM((16 * FPW,), f32),
            pltpu.VMEM((9 * FPW,), f32),
        ],
    )
    recf, colc = setup(vert, facep, nrm, rt)
    raster = pl.kernel(
        _raster_body,
        compiler_params=cparams,
        out_type=(
            jax.ShapeDtypeStruct((3 * HEIGHT * WIDTH,), f32),
            jax.ShapeDtypeStruct((HEIGHT * WIDTH,), f32),
        ),
        mesh=mesh,
        scratch_types=[
            pltpu.VMEM((16 * FPAD,), f32),
            pltpu.VMEM((9 * FPAD,), f32),
            pltpu.VMEM((BAND,), f32),
            pltpu.VMEM((BAND,), i32),
            pltpu.VMEM((3 * BAND,), f32),
            pltpu.VMEM((BAND,), f32),
        ],
    )
    img, msk = raster(recf, colc)
    return img, msk


def kernel(pt_2d, color, pt_3d, normal, R, T, face):
    batch = pt_2d.shape[0]
    images = []
    masks = []
    facep = jnp.zeros((3, FPAD), jnp.int32).at[:, :N_FACES].set(face).reshape(-1)
    for b in range(batch):
        vert = jnp.zeros((9, VPAD), jnp.float32)
        vert = vert.at[0:3, :N_VERTS].set(pt_2d[b])
        vert = vert.at[3:6, :N_VERTS].set(color[b])
        vert = vert.at[6:9, :N_VERTS].set(pt_3d[b])
        vert = vert.reshape(-1)
        nrm = (jnp.zeros((3, FPAD), jnp.float32)
               .at[:, :N_FACES].set(normal[b]).reshape(-1))
        rt = jnp.zeros((16,), jnp.float32).at[0:3].set((R[b].T @ T[b])[:, 0])
        img, msk = _rasterize(vert, facep, nrm, rt)
        images.append(img.reshape(3, HEIGHT, WIDTH))
        masks.append(msk.reshape(HEIGHT, WIDTH))
    image = jnp.stack(images, axis=0)
    mask = jnp.stack(masks, axis=0)
    return image, lax.stop_gradient(mask)


# speedup vs baseline: 1.4795x; 1.3510x over previous
import functools

import jax
import jax.numpy as jnp
import numpy as np
from jax import lax
from jax.experimental import pallas as pl
from jax.experimental.pallas import tpu as pltpu
from jax.experimental.pallas import tpu_sc as plsc

FTINY = float(np.finfo(np.float32).tiny) * 1e3
INF_VALUE = float(np.finfo(np.float32).max) * 1e-3
LOWER_INF = float(np.finfo(np.float32).max) * 1e-4

HEIGHT = 224
WIDTH = 224
N_FACES = 4000
N_VERTS = 2100

NW = 32
L = 16
FPAD = 4096
VPAD = 2112
FPW = FPAD // NW
ROWS = HEIGHT // NW
NCHUNK = WIDTH // L
BAND = ROWS * WIDTH


def _wid():
    return lax.axis_index("s") * 2 + lax.axis_index("c")


def _face_setup_body(vert_hbm, face_hbm, nrm_hbm, rt_hbm,
                     recf_hbm, colc_hbm,
                     vert_v, face_v, nrm_v, rt_v, stf_v, stc_v):
    w = _wid()
    base = w * FPW
    pltpu.sync_copy(vert_hbm, vert_v)
    pltpu.sync_copy(rt_hbm, rt_v)
    for j in range(3):
        pltpu.sync_copy(face_hbm.at[pl.ds(j * FPAD + base, FPW)],
                        face_v.at[pl.ds(j * FPW, FPW)])
        pltpu.sync_copy(nrm_hbm.at[pl.ds(j * FPAD + base, FPW)],
                        nrm_v.at[pl.ds(j * FPW, FPW)])
    rtv = rt_v[pl.ds(0, L)]
    rtx = rtv[0]
    rty = rtv[1]
    rtz = rtv[2]
    iota = lax.iota(jnp.int32, L)

    for k in range(FPW // L):
        o = k * L
        i0 = face_v[pl.ds(o, L)]
        i1 = face_v[pl.ds(FPW + o, L)]
        i2 = face_v[pl.ds(2 * FPW + o, L)]

        def gath(row, idx):
            return plsc.load_gather(vert_v, [jnp.int32(row * VPAD) + idx])

        x0 = gath(0, i0); y0 = gath(1, i0); z0 = gath(2, i0)
        x1 = gath(0, i1); y1 = gath(1, i1); z1 = gath(2, i1)
        x2 = gath(0, i2); y2 = gath(1, i2); z2 = gath(2, i2)

        p3x = gath(6, i0); p3y = gath(7, i0); p3z = gath(8, i0)
        nx = nrm_v[pl.ds(o, L)]
        ny = nrm_v[pl.ds(FPW + o, L)]
        nz = nrm_v[pl.ds(2 * FPW + o, L)]
        ndot = (p3x + rtx) * nx + (p3y + rty) * ny + (p3z + rtz) * nz
        zmin = jnp.minimum(jnp.minimum(z0, z1), z2)
        fid = base + o + iota
        valid = (ndot < 0.0) & (zmin > 0.0) & (fid < N_FACES)

        det = (y1 - y2) * (x0 - x2) + (x2 - x1) * (y0 - y2)
        det = jnp.sign(det) * jnp.maximum(jnp.abs(det), FTINY)
        inv = 1.0 / det
        l0x = (y1 - y2) * inv
        l0y = (x2 - x1) * inv
        l0c = -l0x * x2 - l0y * y2
        l1x = (y2 - y0) * inv
        l1y = (x0 - x2) * inv
        l1c = -l1x * x2 - l1y * y2
        l2x = -l0x - l1x
        l2y = -l0y - l1y
        l2c = 1.0 - l0c - l1c
        dx = z0 * l0x + z1 * l1x + z2 * l2x
        dy = z0 * l0y + z1 * l1y + z2 * l2y
        dc = z0 * l0c + z1 * l1c + z2 * l2c

        rec_off = (o + iota) * 16

        for ch in range(3):
            c0 = gath(3 + ch, i0)
            c1 = gath(3 + ch, i1)
            c2 = gath(3 + ch, i2)
            stc_v[pl.ds((3 * ch + 0) * FPW + o, L)] = c0 * l0x + c1 * l1x + c2 * l2x
            stc_v[pl.ds((3 * ch + 1) * FPW + o, L)] = c0 * l0y + c1 * l1y + c2 * l2y
            stc_v[pl.ds((3 * ch + 2) * FPW + o, L)] = c0 * l0c + c1 * l1c + c2 * l2c

        BIG = 1e9
        span_a, span_b, span_lo, span_hi = [], [], [], []
        for (ex, ey, ec) in ((l0x, l0y, l0c), (l1x, l1y, l1c), (l2x, l2y, l2c)):
            ai = -ec / ex
            bi = -ey / ex
            mag = jnp.abs(ai) + 224.0 * jnp.abs(bi)
            mi = mag * 4e-3 + 1.0
            sane = mag < BIG
            is_lo = (ex > 0.0) & sane
            is_hi = (ex < 0.0) & sane
            span_a.append(jnp.where(is_lo, ai - mi, jnp.where(is_hi, ai + mi, 0.0)))
            span_b.append(jnp.where(is_lo | is_hi, bi, 0.0))
            span_lo.append(is_lo)
            span_hi.append(is_hi)

        ymn = jnp.minimum(jnp.minimum(y0, y1), y2)
        ymx = jnp.maximum(jnp.maximum(y0, y1), y2)
        ymin_i = jnp.clip((ymn - 1.0).astype(jnp.int32), 0, HEIGHT - 1)
        ymax_i = jnp.clip(ymx.astype(jnp.int32) + 1, 0, HEIGHT - 1)
        ymin_i = jnp.where(valid, ymin_i, 1)
        ymax_i = jnp.where(valid, ymax_i, 0)
        packed = ymin_i | (ymax_i << 9)
        for e in range(3):
            packed = (packed
                      | (span_lo[e].astype(jnp.int32) << (18 + 2 * e))
                      | (span_hi[e].astype(jnp.int32) << (19 + 2 * e)))

        vals = (l0x, l0y, l0c, l1x, l1y, l1c, dx, dy, dc,
                span_a[0], span_b[0], span_a[1], span_b[1],
                span_a[2], span_b[2],
                plsc.bitcast(packed, jnp.float32))
        for j, v in enumerate(vals):
            plsc.store_scatter(stf_v, [rec_off + j], v)

    pltpu.sync_copy(stf_v, recf_hbm.at[pl.ds(base * 16, FPW * 16)])
    for j in range(9):
        pltpu.sync_copy(stc_v.at[pl.ds(j * FPW, FPW)],
                        colc_hbm.at[pl.ds(j * FPAD + base, FPW)])


def _raster_body(recf_hbm, colc_hbm,
                 img_hbm, msk_hbm,
                 recf_v, colc_v, zbuf_v, ibuf_v, img_v, msk_v):
    w = _wid()
    band_y0 = w * ROWS
    pltpu.sync_copy(recf_hbm, recf_v)
    pltpu.sync_copy(colc_hbm, colc_v)

    inf_v = jnp.full((L,), INF_VALUE, jnp.float32)
    zero_i = jnp.zeros((L,), jnp.int32)
    for k in range(BAND // L):
        zbuf_v[pl.ds(k * L, L)] = inf_v
        ibuf_v[pl.ds(k * L, L)] = zero_i

    iota_f = lax.iota(jnp.int32, L).astype(jnp.float32)

    def face_body(f, carry):
        rec = recf_v[pl.ds(f * 16, L)]
        reci = plsc.bitcast(rec, jnp.int32)
        pk = reci[15]
        ymin = pk & 511
        ymax = (pk >> 9) & 511
        ylo = jnp.maximum(ymin, band_y0)
        yhi = jnp.minimum(ymax, band_y0 + (ROWS - 1))

        @pl.when(ylo <= yhi)
        def _process():
            def bcast(j):
                return jnp.broadcast_to(rec[j], (L,))

            l0xv = bcast(0)
            l0yv = bcast(1)
            l0cv = bcast(2)
            l1xv = bcast(3)
            l1yv = bcast(4)
            l1cv = bcast(5)
            l2xv = -l0xv - l1xv
            l2yv = -l0yv - l1yv
            l2cv = 1.0 - l0cv - l1cv
            dxv = bcast(6)
            dyv = bcast(7)
            dcv = bcast(8)
            sa0 = rec[9]
            sb0 = rec[10]
            sa1 = rec[11]
            sb1 = rec[12]
            sa2 = rec[13]
            sb2 = rec[14]
            lo0 = (pk & (1 << 18)) != 0
            hi0 = (pk & (1 << 19)) != 0
            lo1 = (pk & (1 << 20)) != 0
            hi1 = (pk & (1 << 21)) != 0
            lo2 = (pk & (1 << 22)) != 0
            hi2 = (pk & (1 << 23)) != 0

            yf0 = band_y0.astype(jnp.float32)
            yf1 = yf0 + float(ROWS - 1)
            xe0a = sa0 + sb0 * yf0
            xe0b = sa0 + sb0 * yf1
            xe1a = sa1 + sb1 * yf0
            xe1b = sa1 + sb1 * yf1
            xe2a = sa2 + sb2 * yf0
            xe2b = sa2 + sb2 * yf1
            xlo = jnp.maximum(
                jnp.maximum(
                    jnp.where(lo0, jnp.minimum(xe0a, xe0b), 0.0),
                    jnp.where(lo1, jnp.minimum(xe1a, xe1b), 0.0)),
                jnp.where(lo2, jnp.minimum(xe2a, xe2b), 0.0))
            xhi = jnp.minimum(
                jnp.minimum(
                    jnp.where(hi0, jnp.maximum(xe0a, xe0b), float(WIDTH - 1)),
                    jnp.where(hi1, jnp.maximum(xe1a, xe1b), float(WIDTH - 1))),
                jnp.where(hi2, jnp.maximum(xe2a, xe2b), float(WIDTH - 1)))
            v0 = xlo * (1.0 / L)
            v1 = xhi * (1.0 / L)
            t0 = v0.astype(jnp.int32)
            t1 = v1.astype(jnp.int32)
            t0 = t0 - (t0.astype(jnp.float32) > v0).astype(jnp.int32)
            t1 = t1 - (t1.astype(jnp.float32) > v1).astype(jnp.int32)
            c0r = jnp.clip(t0, 0, NCHUNK - 1)
            c1r = jnp.clip(t1, 0, NCHUNK - 1)

            a0v = l0yv * yf0 + l0cv
            a1v = l1yv * yf0 + l1cv
            a2v = l2yv * yf0 + l2cv
            adv = dyv * yf0 + dcv

            def col_body(cix, _c):
                xv = (cix * L).astype(jnp.float32) + iota_f
                lv0 = l0xv * xv + a0v
                lv1 = l1xv * xv + a1v
                lv2 = l2xv * xv + a2v
                d = dxv * xv + adv
                off0 = cix * L
                for r in range(ROWS):
                    if r:
                        lv0 = lv0 + l0yv
                        lv1 = lv1 + l1yv
                        lv2 = lv2 + l2yv
                        d = d + dyv
                    m = jnp.minimum(jnp.minimum(lv0, lv1), lv2) >= 0.0
                    off = off0 + r * WIDTH
                    z = zbuf_v[pl.ds(off, L)]
                    upd = m & (d < z)
                    zbuf_v[pl.ds(off, L)] = jnp.where(upd, d, z)
                    ib = ibuf_v[pl.ds(off, L)]
                    ibuf_v[pl.ds(off, L)] = jnp.where(upd, f, ib)
                return 0

            lax.fori_loop(c0r, c1r + 1, col_body, 0)

        return carry

    lax.fori_loop(0, FPAD, face_body, 0)

    def res_body(k, _):
        off = k * L
        z = zbuf_v[pl.ds(off, L)]
        idx = ibuf_v[pl.ds(off, L)]
        mskf = (z < LOWER_INF).astype(jnp.float32)
        y = band_y0 + k // NCHUNK
        yf = y.astype(jnp.float32)
        xv = ((k % NCHUNK) * L).astype(jnp.float32) + iota_f
        for ch in range(3):
            cx = plsc.load_gather(colc_v, [jnp.int32((3 * ch + 0) * FPAD) + idx])
            cy = plsc.load_gather(colc_v, [jnp.int32((3 * ch + 1) * FPAD) + idx])
            cc = plsc.load_gather(colc_v, [jnp.int32((3 * ch + 2) * FPAD) + idx])
            img_v[pl.ds(ch * BAND + off, L)] = mskf * (cx * xv + cy * yf + cc)
        msk_v[pl.ds(off, L)] = mskf
        return 0

    lax.fori_loop(0, BAND // L, res_body, 0)

    for ch in range(3):
        pltpu.sync_copy(img_v.at[pl.ds(ch * BAND, BAND)],
                        img_hbm.at[pl.ds(ch * (HEIGHT * WIDTH) + w * BAND, BAND)])
    pltpu.sync_copy(msk_v, msk_hbm.at[pl.ds(w * BAND, BAND)])


def _build_mesh():
    return plsc.VectorSubcoreMesh(core_axis_name="c", subcore_axis_name="s")


@jax.jit
def _rasterize(vert, facep, nrm, rt):
    mesh = _build_mesh()
    f32 = jnp.float32
    i32 = jnp.int32
    cparams = pltpu.CompilerParams(needs_layout_passes=False)
    setup = pl.kernel(
        _face_setup_body,
        compiler_params=cparams,
        out_type=(
            jax.ShapeDtypeStruct((16 * FPAD,), f32),
            jax.ShapeDtypeStruct((9 * FPAD,), f32),
        ),
        mesh=mesh,
        scratch_types=[
            pltpu.VMEM((9 * VPAD,), f32),
            pltpu.VMEM((3 * FPW,), i32),
            pltpu.VMEM((3 * FPW,), f32),
            pltpu.VMEM((16,), f32),
            pltpu.VMEM((16 * FPW,), f32),
            pltpu.VMEM((9 * FPW,), f32),
        ],
    )
    recf, colc = setup(vert, facep, nrm, rt)
    raster = pl.kernel(
        _raster_body,
        compiler_params=cparams,
        out_type=(
            jax.ShapeDtypeStruct((3 * HEIGHT * WIDTH,), f32),
            jax.ShapeDtypeStruct((HEIGHT * WIDTH,), f32),
        ),
        mesh=mesh,
        scratch_types=[
            pltpu.VMEM((16 * FPAD,), f32),
            pltpu.VMEM((9 * FPAD,), f32),
            pltpu.VMEM((BAND,), f32),
            pltpu.VMEM((BAND,), i32),
            pltpu.VMEM((3 * BAND,), f32),
            pltpu.VMEM((BAND,), f32),
        ],
    )
    img, msk = raster(recf, colc)
    return img, msk


def kernel(pt_2d, color, pt_3d, normal, R, T, face):
    batch = pt_2d.shape[0]
    images = []
    masks = []
    facep = jnp.zeros((3, FPAD), jnp.int32).at[:, :N_FACES].set(face).reshape(-1)
    for b in range(batch):
        vert = jnp.zeros((9, VPAD), jnp.float32)
        vert = vert.at[0:3, :N_VERTS].set(pt_2d[b])
        vert = vert.at[3:6, :N_VERTS].set(color[b])
        vert = vert.at[6:9, :N_VERTS].set(pt_3d[b])
        vert = vert.reshape(-1)
        nrm = (jnp.zeros((3, FPAD), jnp.float32)
               .at[:, :N_FACES].set(normal[b]).reshape(-1))
        rt = jnp.zeros((16,), jnp.float32).at[0:3].set((R[b].T @ T[b])[:, 0])
        img, msk = _rasterize(vert, facep, nrm, rt)
        images.append(img.reshape(3, HEIGHT, WIDTH))
        masks.append(msk.reshape(HEIGHT, WIDTH))
    image = jnp.stack(images, axis=0)
    mask = jnp.stack(masks, axis=0)
    return image, lax.stop_gradient(mask)
